# baseline (speedup 1.0000x reference)
import functools
import numpy as np
import jax
import jax.numpy as jnp
from jax import lax
from jax.experimental import pallas as pl
from jax.experimental.pallas import tpu as pltpu
from jax.experimental.pallas import tpu_sc as plsc

N = 10000
E = 320000
F = 128
D = 128
DH = D // 2

NC = 2
NS = 16
EPS = E // NS
CK = 80
NCH = EPS // CK
CKE = 40
NCHE = E // (NC * NS * CKE)
N2 = 10240
RPT2 = N2 // NS
RPT = N // NS

BM = 1000
GRID = N // BM

_mesh = plsc.VectorSubcoreMesh(core_axis_name="c", subcore_axis_name="s")


@functools.partial(
    pl.kernel,
    out_type=jax.ShapeDtypeStruct((NC * 2 * N2,), jnp.float32),
    mesh=_mesh,
    compiler_params=pltpu.CompilerParams(use_tc_tiling_on_sc=False),
    scratch_types=[
        pltpu.VMEM((NCH, CK), jnp.int32),
        pltpu.VMEM((NCH, CK), jnp.int32),
        pltpu.VMEM((CK,), jnp.float32),
        pltpu.VMEM_SHARED((N2,), jnp.float32),
        pltpu.VMEM_SHARED((N2,), jnp.float32),
        pltpu.SemaphoreType.DMA,
        pltpu.SemaphoreType.DMA,
    ],
)
def _degrees(srcr_hbm, dstr_hbm, zeros1_hbm, out_hbm,
             src_v, dst_v, ones_v, acca_sh, accb_sh, sema, semb):
    c = lax.axis_index("c")
    s = lax.axis_index("s")
    for i in range(CK // 16):
        ones_v[pl.ds(i * 16, 16)] = jnp.ones((16,), jnp.float32)
    pltpu.sync_copy(srcr_hbm.at[s], src_v)
    pltpu.sync_copy(dstr_hbm.at[s], dst_v)
    pltpu.sync_copy(zeros1_hbm, acca_sh.at[pl.ds(s * RPT2, RPT2)])
    pltpu.sync_copy(zeros1_hbm, accb_sh.at[pl.ds(s * RPT2, RPT2)])
    plsc.subcore_barrier()

    half = NCH // 2
    W = 8

    def body(j, carry):
        jj = c * half + j
        pltpu.async_copy(ones_v, acca_sh.at[src_v.at[jj]], sema, add=True)
        pltpu.async_copy(ones_v, accb_sh.at[dst_v.at[jj]], semb, add=True)

        @pl.when(j >= W)
        def _():
            jd = jj - W
            pltpu.make_async_copy(ones_v, acca_sh.at[src_v.at[jd]],
                                  sema).wait()
            pltpu.make_async_copy(ones_v, accb_sh.at[dst_v.at[jd]],
                                  semb).wait()
        return carry

    lax.fori_loop(0, half, body, 0)
    for w in range(W):
        jd = c * half + (half - W + w)
        pltpu.make_async_copy(ones_v, acca_sh.at[src_v.at[jd]], sema).wait()
        pltpu.make_async_copy(ones_v, accb_sh.at[dst_v.at[jd]], semb).wait()
    plsc.subcore_barrier()
    pltpu.sync_copy(acca_sh.at[pl.ds(s * RPT2, RPT2)],
                    out_hbm.at[pl.ds((c * 2 + 0) * N2 + s * RPT2, RPT2)])
    pltpu.sync_copy(accb_sh.at[pl.ds(s * RPT2, RPT2)],
                    out_hbm.at[pl.ds((c * 2 + 1) * N2 + s * RPT2, RPT2)])


@functools.partial(
    pl.kernel,
    out_type=jax.ShapeDtypeStruct((NC, N, D), jnp.float32),
    mesh=_mesh,
    compiler_params=pltpu.CompilerParams(use_tc_tiling_on_sc=False),
    scratch_types=[
        pltpu.VMEM((NCHE, CKE), jnp.int32),
        pltpu.VMEM((NCHE, CKE), jnp.int32),
        [pltpu.VMEM((CKE, D), jnp.float32) for _ in range(4)],
        pltpu.VMEM_SHARED((N, D), jnp.float32),
        [pltpu.SemaphoreType.DMA for _ in range(4)],
    ],
)
def _spmm(y_hbm, srcr_hbm, dstr_hbm, zeros_hbm, out_hbm,
          src_v, dst_v, rows, acc_sh, sems):
    c = lax.axis_index("c")
    s = lax.axis_index("s")
    pltpu.sync_copy(srcr_hbm.at[c, s], src_v)
    pltpu.sync_copy(dstr_hbm.at[c, s], dst_v)

    R = 4
    A = R // 2
    for u in range(A):
        pltpu.async_copy(y_hbm.at[src_v.at[u]], rows[u], sems[u])

    pltpu.sync_copy(zeros_hbm, acc_sh.at[pl.ds(s * RPT, RPT)])
    plsc.subcore_barrier()

    def body(jj, carry):
        base = R * jj
        for u in range(R):
            k = base + u
            pltpu.make_async_copy(y_hbm.at[src_v.at[k]], rows[u],
                                  sems[u]).wait()
            pltpu.async_copy(rows[u], acc_sh.at[dst_v.at[k]], sems[u],
                             add=True)
            v = (u + A) % R
            kf = k - A

            @pl.when(kf >= 0)
            def _():
                pltpu.make_async_copy(rows[v], acc_sh.at[dst_v.at[kf]],
                                      sems[v]).wait()

            @pl.when(k + A < NCHE)
            def _():
                pltpu.async_copy(y_hbm.at[src_v.at[k + A]], rows[v], sems[v])
        return carry

    lax.fori_loop(0, NCHE // R, body, 0)
    tail = NCHE - (NCHE // R) * R
    for k in range(NCHE - tail, NCHE):
        u = k % R
        pltpu.make_async_copy(y_hbm.at[src_v.at[k]], rows[u], sems[u]).wait()
        pltpu.async_copy(rows[u], acc_sh.at[dst_v.at[k]], sems[u], add=True)
    for k in range(NCHE - (A + tail), NCHE):
        u = k % R
        pltpu.make_async_copy(rows[u], acc_sh.at[dst_v.at[k]], sems[u]).wait()

    plsc.subcore_barrier()
    pltpu.sync_copy(acc_sh.at[pl.ds(s * RPT, RPT)],
                    out_hbm.at[c, pl.ds(s * RPT, RPT)])


def _matmul1_body(x_ref, w1_ref, y_ref):
    y_ref[...] = jnp.dot(x_ref[...], w1_ref[...],
                         preferred_element_type=jnp.float32)


def _norm1_body(yp_ref, degp_ref, y1_ref, ns_ref, nd_ref):
    dp = degp_ref[...]
    dsum = dp[0] + dp[1]
    ns = lax.rsqrt(jnp.maximum(dsum[0], 1.0))
    nd = lax.rsqrt(jnp.maximum(dsum[1], 1.0))
    y1_ref[...] = yp_ref[...] * ns
    ns_ref[...] = ns
    nd_ref[...] = nd


def _dense2_body(aggp_ref, ns_ref, nd_ref, b1_ref, w2_ref, y2_ref):
    agg = aggp_ref[0] + aggp_ref[1]
    h1 = jnp.maximum(agg * nd_ref[...] + b1_ref[...], 0.0)
    y2_ref[...] = jnp.dot(h1 * ns_ref[...], w2_ref[...],
                          preferred_element_type=jnp.float32)


def _final_body(aggp_ref, nd_ref, b2_ref, emb_ref, acc_h, acc_n):
    i = pl.program_id(0)
    agg = aggp_ref[0] + aggp_ref[1]
    h2 = agg * nd_ref[...] + b2_ref[...]
    rs = jnp.sum(h2, axis=0, keepdims=True)
    rn = jnp.sum(jnp.sqrt(jnp.sum(h2 * h2, axis=1, keepdims=True)))

    @pl.when(i == 0)
    def _():
        acc_h[...] = rs
        acc_n[0, 0] = rn

    @pl.when(i > 0)
    def _():
        acc_h[...] = acc_h[...] + rs
        acc_n[0, 0] = acc_n[0, 0] + rn

    @pl.when(i == GRID - 1)
    def _():
        emb_ref[...] = acc_h[...] * (np.sqrt(float(D)) * N / acc_n[0, 0])


_matmul1 = pl.pallas_call(
    _matmul1_body,
    grid=(GRID,),
    in_specs=[
        pl.BlockSpec((BM, F), lambda i: (i, 0)),
        pl.BlockSpec((F, D), lambda i: (0, 0)),
    ],
    out_specs=pl.BlockSpec((BM, D), lambda i: (i, 0)),
    out_shape=jax.ShapeDtypeStruct((N, D), jnp.float32),
)

_norm1 = pl.pallas_call(
    _norm1_body,
    grid=(GRID,),
    in_specs=[
        pl.BlockSpec((BM, D), lambda i: (i, 0)),
        pl.BlockSpec((NC, 2, BM, 1), lambda i: (0, 0, i, 0)),
    ],
    out_specs=[
        pl.BlockSpec((BM, D), lambda i: (i, 0)),
        pl.BlockSpec((BM, 1), lambda i: (i, 0)),
        pl.BlockSpec((BM, 1), lambda i: (i, 0)),
    ],
    out_shape=[
        jax.ShapeDtypeStruct((N, D), jnp.float32),
        jax.ShapeDtypeStruct((N, 1), jnp.float32),
        jax.ShapeDtypeStruct((N, 1), jnp.float32),
    ],
)

_dense2 = pl.pallas_call(
    _dense2_body,
    grid=(GRID,),
    in_specs=[
        pl.BlockSpec((NC, BM, D), lambda i: (0, i, 0)),
        pl.BlockSpec((BM, 1), lambda i: (i, 0)),
        pl.BlockSpec((BM, 1), lambda i: (i, 0)),
        pl.BlockSpec((1, D), lambda i: (0, 0)),
        pl.BlockSpec((D, D), lambda i: (0, 0)),
    ],
    out_specs=pl.BlockSpec((BM, D), lambda i: (i, 0)),
    out_shape=jax.ShapeDtypeStruct((N, D), jnp.float32),
)

_final = pl.pallas_call(
    _final_body,
    grid=(GRID,),
    in_specs=[
        pl.BlockSpec((NC, BM, D), lambda i: (0, i, 0)),
        pl.BlockSpec((BM, 1), lambda i: (i, 0)),
        pl.BlockSpec((1, D), lambda i: (0, 0)),
    ],
    out_specs=pl.BlockSpec((1, D), lambda i: (0, 0)),
    out_shape=jax.ShapeDtypeStruct((1, D), jnp.float32),
    scratch_shapes=[
        pltpu.VMEM((1, D), jnp.float32),
        pltpu.SMEM((1, 1), jnp.float32),
    ],
)


def kernel(x, edge_index, W1, b1, W2, b2):
    src = edge_index[0].reshape(NS, NCH, CK)
    dst = edge_index[1].reshape(NS, NCH, CK)
    srce = edge_index[0].reshape(NC, NS, NCHE, CKE)
    dste = edge_index[1].reshape(NC, NS, NCHE, CKE)
    z1 = jnp.zeros((RPT2,), jnp.float32)
    z2 = jnp.zeros((RPT, D), jnp.float32)

    degp = _degrees(src, dst, z1).reshape(NC, 2, N2, 1)[:, :, :N, :]
    y = _matmul1(x, W1)
    y1, ns, nd = _norm1(y, degp)
    agg1 = _spmm(y1, srce, dste, z2)
    y2 = _dense2(agg1, ns, nd, b1.reshape(1, D), W2)
    agg2 = _spmm(y2, srce, dste, z2)
    emb = _final(agg2, nd, b2.reshape(1, D))
    return emb


# speedup vs baseline: 1.0959x; 1.0959x over previous
import functools
import numpy as np
import jax
import jax.numpy as jnp
from jax import lax
from jax.experimental import pallas as pl
from jax.experimental.pallas import tpu as pltpu
from jax.experimental.pallas import tpu_sc as plsc

N = 10000
E = 320000
F = 128
D = 128
DH = D // 2

NC = 2
NS = 16
EPS = E // NS
CK = 80
NCH = EPS // CK
N2 = 10240
RPT2 = N2 // NS
RPT = N // NS

BM = 2000
GRID = N // BM

_mesh = plsc.VectorSubcoreMesh(core_axis_name="c", subcore_axis_name="s")


@functools.partial(
    pl.kernel,
    out_type=jax.ShapeDtypeStruct((NC * 2 * N2,), jnp.float32),
    mesh=_mesh,
    compiler_params=pltpu.CompilerParams(use_tc_tiling_on_sc=False),
    scratch_types=[
        pltpu.VMEM((NCH, CK), jnp.int32),
        pltpu.VMEM((NCH, CK), jnp.int32),
        pltpu.VMEM((CK,), jnp.float32),
        pltpu.VMEM_SHARED((N2,), jnp.float32),
        pltpu.VMEM_SHARED((N2,), jnp.float32),
        pltpu.SemaphoreType.DMA,
        pltpu.SemaphoreType.DMA,
    ],
)
def _degrees(srcr_hbm, dstr_hbm, zeros1_hbm, out_hbm,
             src_v, dst_v, ones_v, acca_sh, accb_sh, sema, semb):
    c = lax.axis_index("c")
    s = lax.axis_index("s")
    for i in range(CK // 16):
        ones_v[pl.ds(i * 16, 16)] = jnp.ones((16,), jnp.float32)
    pltpu.sync_copy(srcr_hbm.at[s], src_v)
    pltpu.sync_copy(dstr_hbm.at[s], dst_v)
    pltpu.sync_copy(zeros1_hbm, acca_sh.at[pl.ds(s * RPT2, RPT2)])
    pltpu.sync_copy(zeros1_hbm, accb_sh.at[pl.ds(s * RPT2, RPT2)])
    plsc.subcore_barrier()

    half = NCH // 2
    W = 8

    def body(j, carry):
        jj = c * half + j
        pltpu.async_copy(ones_v, acca_sh.at[src_v.at[jj]], sema, add=True)
        pltpu.async_copy(ones_v, accb_sh.at[dst_v.at[jj]], semb, add=True)

        @pl.when(j >= W)
        def _():
            jd = jj - W
            pltpu.make_async_copy(ones_v, acca_sh.at[src_v.at[jd]],
                                  sema).wait()
            pltpu.make_async_copy(ones_v, accb_sh.at[dst_v.at[jd]],
                                  semb).wait()
        return carry

    lax.fori_loop(0, half, body, 0)
    for w in range(W):
        jd = c * half + (half - W + w)
        pltpu.make_async_copy(ones_v, acca_sh.at[src_v.at[jd]], sema).wait()
        pltpu.make_async_copy(ones_v, accb_sh.at[dst_v.at[jd]], semb).wait()
    plsc.subcore_barrier()
    pltpu.sync_copy(acca_sh.at[pl.ds(s * RPT2, RPT2)],
                    out_hbm.at[pl.ds((c * 2 + 0) * N2 + s * RPT2, RPT2)])
    pltpu.sync_copy(accb_sh.at[pl.ds(s * RPT2, RPT2)],
                    out_hbm.at[pl.ds((c * 2 + 1) * N2 + s * RPT2, RPT2)])


@functools.partial(
    pl.kernel,
    out_type=jax.ShapeDtypeStruct((NC, N, DH), jnp.float32),
    mesh=_mesh,
    compiler_params=pltpu.CompilerParams(use_tc_tiling_on_sc=False),
    scratch_types=[
        pltpu.VMEM((NCH, CK), jnp.int32),
        pltpu.VMEM((NCH, CK), jnp.int32),
        [pltpu.VMEM((CK, DH), jnp.float32) for _ in range(8)],
        pltpu.VMEM_SHARED((N, DH), jnp.float32),
        [pltpu.SemaphoreType.DMA for _ in range(8)],
    ],
)
def _spmm(y_hbm, srcr_hbm, dstr_hbm, zeros_hbm, out_hbm,
          src_v, dst_v, rows, acc_sh, sems):
    c = lax.axis_index("c")
    s = lax.axis_index("s")
    pltpu.sync_copy(srcr_hbm.at[s], src_v)
    pltpu.sync_copy(dstr_hbm.at[s], dst_v)

    ytab = y_hbm.at[c]

    R = 8
    A = R // 2
    for u in range(A):
        pltpu.async_copy(ytab.at[src_v.at[u]], rows[u], sems[u])

    pltpu.sync_copy(zeros_hbm, acc_sh.at[pl.ds(s * RPT, RPT)])
    plsc.subcore_barrier()

    def body(jj, carry):
        base = R * jj
        for u in range(R):
            k = base + u
            pltpu.make_async_copy(ytab.at[src_v.at[k]], rows[u],
                                  sems[u]).wait()
            pltpu.async_copy(rows[u], acc_sh.at[dst_v.at[k]], sems[u],
                             add=True)
            v = (u + A) % R
            kf = k - A

            @pl.when(kf >= 0)
            def _():
                pltpu.make_async_copy(rows[v], acc_sh.at[dst_v.at[kf]],
                                      sems[v]).wait()

            @pl.when(k + A < NCH)
            def _():
                pltpu.async_copy(ytab.at[src_v.at[k + A]], rows[v], sems[v])
        return carry

    lax.fori_loop(0, NCH // R, body, 0)
    tail = NCH - (NCH // R) * R
    for k in range(NCH - tail, NCH):
        u = k % R
        pltpu.make_async_copy(ytab.at[src_v.at[k]], rows[u], sems[u]).wait()
        pltpu.async_copy(rows[u], acc_sh.at[dst_v.at[k]], sems[u], add=True)
    for k in range(NCH - (A + tail), NCH):
        u = k % R
        pltpu.make_async_copy(rows[u], acc_sh.at[dst_v.at[k]], sems[u]).wait()

    plsc.subcore_barrier()
    pltpu.sync_copy(acc_sh.at[pl.ds(s * RPT, RPT)],
                    out_hbm.at[c, pl.ds(s * RPT, RPT)])


def _matmul1_body(x_ref, w1_ref, y_ref):
    y = jnp.dot(x_ref[...].astype(jnp.bfloat16), w1_ref[...],
                preferred_element_type=jnp.float32)
    y_ref[0] = y[:, :DH]
    y_ref[1] = y[:, DH:]


def _norm1_body(yp_ref, degp_ref, y1_ref, ns_ref, nd_ref):
    dp = degp_ref[...]
    dsum = dp[0] + dp[1]
    ns = lax.rsqrt(jnp.maximum(dsum[0], 1.0))
    nd = lax.rsqrt(jnp.maximum(dsum[1], 1.0))
    y1_ref[0] = yp_ref[0] * ns
    y1_ref[1] = yp_ref[1] * ns
    ns_ref[...] = ns
    nd_ref[...] = nd


def _dense2_body(aggp_ref, ns_ref, nd_ref, b1_ref, w2_ref, y2_ref):
    agg = jnp.concatenate([aggp_ref[0], aggp_ref[1]], axis=1)
    h1 = jnp.maximum(agg * nd_ref[...] + b1_ref[...], 0.0)
    y = jnp.dot((h1 * ns_ref[...]).astype(jnp.bfloat16), w2_ref[...],
                preferred_element_type=jnp.float32)
    y2_ref[0] = y[:, :DH]
    y2_ref[1] = y[:, DH:]


def _final_body(aggp_ref, nd_ref, b2_ref, emb_ref, acc_h, acc_n):
    i = pl.program_id(0)
    agg = jnp.concatenate([aggp_ref[0], aggp_ref[1]], axis=1)
    h2 = agg * nd_ref[...] + b2_ref[...]
    rs = jnp.sum(h2, axis=0, keepdims=True)
    rn = jnp.sum(jnp.sqrt(jnp.sum(h2 * h2, axis=1, keepdims=True)))

    @pl.when(i == 0)
    def _():
        acc_h[...] = rs
        acc_n[0, 0] = rn

    @pl.when(i > 0)
    def _():
        acc_h[...] = acc_h[...] + rs
        acc_n[0, 0] = acc_n[0, 0] + rn

    @pl.when(i == GRID - 1)
    def _():
        emb_ref[...] = acc_h[...] * (np.sqrt(float(D)) * N / acc_n[0, 0])


_matmul1 = pl.pallas_call(
    _matmul1_body,
    grid=(GRID,),
    in_specs=[
        pl.BlockSpec((BM, F), lambda i: (i, 0)),
        pl.BlockSpec((F, D), lambda i: (0, 0)),
    ],
    out_specs=pl.BlockSpec((NC, BM, DH), lambda i: (0, i, 0)),
    out_shape=jax.ShapeDtypeStruct((NC, N, DH), jnp.float32),
)

_norm1 = pl.pallas_call(
    _norm1_body,
    grid=(GRID,),
    in_specs=[
        pl.BlockSpec((NC, BM, DH), lambda i: (0, i, 0)),
        pl.BlockSpec((NC, 2, BM, 1), lambda i: (0, 0, i, 0)),
    ],
    out_specs=[
        pl.BlockSpec((NC, BM, DH), lambda i: (0, i, 0)),
        pl.BlockSpec((BM, 1), lambda i: (i, 0)),
        pl.BlockSpec((BM, 1), lambda i: (i, 0)),
    ],
    out_shape=[
        jax.ShapeDtypeStruct((NC, N, DH), jnp.float32),
        jax.ShapeDtypeStruct((N, 1), jnp.float32),
        jax.ShapeDtypeStruct((N, 1), jnp.float32),
    ],
)

_dense2 = pl.pallas_call(
    _dense2_body,
    grid=(GRID,),
    in_specs=[
        pl.BlockSpec((NC, BM, DH), lambda i: (0, i, 0)),
        pl.BlockSpec((BM, 1), lambda i: (i, 0)),
        pl.BlockSpec((BM, 1), lambda i: (i, 0)),
        pl.BlockSpec((1, D), lambda i: (0, 0)),
        pl.BlockSpec((D, D), lambda i: (0, 0)),
    ],
    out_specs=pl.BlockSpec((NC, BM, DH), lambda i: (0, i, 0)),
    out_shape=jax.ShapeDtypeStruct((NC, N, DH), jnp.float32),
)

_final = pl.pallas_call(
    _final_body,
    grid=(GRID,),
    in_specs=[
        pl.BlockSpec((NC, BM, DH), lambda i: (0, i, 0)),
        pl.BlockSpec((BM, 1), lambda i: (i, 0)),
        pl.BlockSpec((1, D), lambda i: (0, 0)),
    ],
    out_specs=pl.BlockSpec((1, D), lambda i: (0, 0)),
    out_shape=jax.ShapeDtypeStruct((1, D), jnp.float32),
    scratch_shapes=[
        pltpu.VMEM((1, D), jnp.float32),
        pltpu.SMEM((1, 1), jnp.float32),
    ],
)


def kernel(x, edge_index, W1, b1, W2, b2):
    src = edge_index[0].reshape(NS, NCH, CK)
    dst = edge_index[1].reshape(NS, NCH, CK)
    z1 = jnp.zeros((RPT2,), jnp.float32)
    z2 = jnp.zeros((RPT, DH), jnp.float32)

    degp = _degrees(src, dst, z1).reshape(NC, 2, N2, 1)[:, :, :N, :]
    y = _matmul1(x, W1.astype(jnp.bfloat16))
    y1, ns, nd = _norm1(y, degp)
    agg1 = _spmm(y1, src, dst, z2)
    y2 = _dense2(agg1, ns, nd, b1.reshape(1, D), W2.astype(jnp.bfloat16))
    agg2 = _spmm(y2, src, dst, z2)
    emb = _final(agg2, nd, b2.reshape(1, D))
    return emb
